# Initial kernel scaffold; baseline (speedup 1.0000x reference)
import functools

import jax
import jax.numpy as jnp
from jax import lax
from jax.experimental import pallas as pl
from jax.experimental.pallas import tpu as pltpu
from jax.experimental.pallas import tpu_sc as plsc

N = 10000
E = 320000
F_IN = 128
C = 40
CP = 48
DW = 16

NC, NS = 2, 16
NT = NC * NS
CH = 128
KS = 8
SUP = CH * KS

NPAD = 10240
DUMMY = NPAD - 1
EPT = 10240
EPAD = EPT * NT
NSUP = EPT // SUP
ROWS_PER_SUB = NPAD // NS

_MESH = plsc.VectorSubcoreMesh(core_axis_name="c", subcore_axis_name="s")


@functools.partial(
    pl.kernel,
    out_type=jax.ShapeDtypeStruct((NC, NPAD, DW), jnp.float32),
    mesh=_MESH,
    scratch_types=[
        pltpu.VMEM((KS, CH), jnp.int32),
        pltpu.VMEM((CH, DW), jnp.float32),
        pltpu.VMEM_SHARED((NPAD, DW), jnp.float32),
        pltpu.SemaphoreType.DMA,
    ],
)
def _sc_degree(col_hbm, ones_hbm, zeros_hbm, out_hbm, idxc, onesv, acc, sem):
    c = lax.axis_index("c")
    s = lax.axis_index("s")
    tid = c * NS + s
    r0 = s * ROWS_PER_SUB
    pltpu.sync_copy(zeros_hbm.at[pl.ds(r0, ROWS_PER_SUB)],
                    acc.at[pl.ds(r0, ROWS_PER_SUB)])
    pltpu.sync_copy(ones_hbm, onesv)
    plsc.subcore_barrier()

    @pl.loop(0, NSUP)
    def _(k):
        base = tid * (EPT // CH) + k * KS
        pltpu.sync_copy(col_hbm.at[pl.ds(base, KS)], idxc)
        copies = [
            pltpu.async_copy(onesv, acc.at[idxc.at[j]], sem, add=True)
            for j in range(KS)
        ]
        for cp in copies:
            cp.wait()

    plsc.subcore_barrier()
    pltpu.sync_copy(acc.at[pl.ds(r0, ROWS_PER_SUB)],
                    out_hbm.at[c].at[pl.ds(r0, ROWS_PER_SUB)])


@functools.partial(
    pl.kernel,
    out_type=jax.ShapeDtypeStruct((NC, NPAD, CP), jnp.float32),
    mesh=_MESH,
    scratch_types=[
        pltpu.VMEM((KS, CH), jnp.int32),
        pltpu.VMEM((KS, CH), jnp.int32),
        pltpu.VMEM((KS, CH, CP), jnp.float32),
        pltpu.VMEM_SHARED((NPAD, CP), jnp.float32),
        pltpu.SemaphoreType.DMA,
    ],
)
def _sc_hop(g_hbm, row_hbm, col_hbm, zeros_hbm, out_hbm,
            idxr, idxc, vals, acc, sem):
    c = lax.axis_index("c")
    s = lax.axis_index("s")
    tid = c * NS + s
    r0 = s * ROWS_PER_SUB
    pltpu.sync_copy(zeros_hbm.at[pl.ds(r0, ROWS_PER_SUB)],
                    acc.at[pl.ds(r0, ROWS_PER_SUB)])
    plsc.subcore_barrier()

    @pl.loop(0, NSUP)
    def _(k):
        base = tid * (EPT // CH) + k * KS
        pltpu.sync_copy(row_hbm.at[pl.ds(base, KS)], idxr)
        pltpu.sync_copy(col_hbm.at[pl.ds(base, KS)], idxc)
        gathers = [
            pltpu.async_copy(g_hbm.at[idxr.at[j]], vals.at[j], sem)
            for j in range(KS)
        ]
        for cp in gathers:
            cp.wait()
        scatters = [
            pltpu.async_copy(vals.at[j], acc.at[idxc.at[j]], sem, add=True)
            for j in range(KS)
        ]
        for cp in scatters:
            cp.wait()

    plsc.subcore_barrier()
    pltpu.sync_copy(acc.at[pl.ds(r0, ROWS_PER_SUB)],
                    out_hbm.at[c].at[pl.ds(r0, ROWS_PER_SUB)])


def _tc_matmul(xp, wt):
    def body(x_ref, w_ref, o_ref):
        o_ref[...] = jnp.dot(x_ref[...], w_ref[...],
                             preferred_element_type=jnp.float32)
    return pl.pallas_call(
        body, out_shape=jax.ShapeDtypeStruct((NPAD, CP), jnp.float32),
    )(xp, wt)


def _deg_col(d_ref):
    return d_ref[0, :, 0:1] + d_ref[1, :, 0:1] + 1.0


def _tc_scale(degp, y):
    def body(d_ref, y_ref, o_ref):
        o_ref[...] = y_ref[...] * lax.rsqrt(_deg_col(d_ref))
    return pl.pallas_call(
        body, out_shape=jax.ShapeDtypeStruct((NPAD, CP), jnp.float32),
    )(degp, y)


def _tc_mid(p, g0, degp):
    def body(p_ref, g_ref, d_ref, o_ref):
        s = p_ref[0] + p_ref[1] + g_ref[...]
        o_ref[...] = s / _deg_col(d_ref)
    return pl.pallas_call(
        body, out_shape=jax.ShapeDtypeStruct((NPAD, CP), jnp.float32),
    )(p, g0, degp)


def _tc_final(q, g1, degp, bp):
    def body(q_ref, g_ref, d_ref, b_ref, o_ref):
        t = (q_ref[0] + q_ref[1] + g_ref[...]) * lax.rsqrt(_deg_col(d_ref))
        logits = t + b_ref[...]
        mask = lax.broadcasted_iota(jnp.int32, (1, CP), 1) < C
        masked = jnp.where(mask, logits, -jnp.inf)
        m = jnp.max(masked, axis=1, keepdims=True)
        ex = jnp.where(mask, jnp.exp(logits - m), 0.0)
        lse = jnp.log(jnp.sum(ex, axis=1, keepdims=True))
        o_ref[...] = logits - m - lse
    return pl.pallas_call(
        body, out_shape=jax.ShapeDtypeStruct((NPAD, CP), jnp.float32),
    )(q, g1, degp, bp)


def kernel(x, edge_index, W, b):
    f32 = jnp.float32
    xp = jnp.zeros((NPAD, F_IN), f32).at[:N].set(x)
    wt = jnp.zeros((F_IN, CP), f32).at[:, :C].set(W.T)
    bp = jnp.zeros((1, CP), f32).at[0, :C].set(b)

    pad_r = jnp.zeros((EPAD - E,), jnp.int32)
    pad_c = jnp.full((EPAD - E,), DUMMY, jnp.int32)
    rows = jnp.concatenate([edge_index[0].astype(jnp.int32), pad_r])
    cols = jnp.concatenate([edge_index[1].astype(jnp.int32), pad_c])
    rows2d = rows.reshape(EPAD // CH, CH)
    cols2d = cols.reshape(EPAD // CH, CH)

    zeros_dw = jnp.zeros((NPAD, DW), f32)
    zeros_cp = jnp.zeros((NPAD, CP), f32)
    ones_ch = jnp.ones((CH, DW), f32)

    degp = _sc_degree(cols2d, ones_ch, zeros_dw)
    y = _tc_matmul(xp, wt)
    g0 = _tc_scale(degp, y)
    p = _sc_hop(g0, rows2d, cols2d, zeros_cp)
    g1 = _tc_mid(p, g0, degp)
    q = _sc_hop(g1, rows2d, cols2d, zeros_cp)
    out = _tc_final(q, g1, degp, bp)
    return out[:N, :C]


# speedup vs baseline: 18.6636x; 18.6636x over previous
import functools

import jax
import jax.numpy as jnp
from jax import lax
from jax.experimental import pallas as pl
from jax.experimental.pallas import tpu as pltpu
from jax.experimental.pallas import tpu_sc as plsc

N = 10000
E = 320000
F_IN = 128
C = 40
CP = 48
DW = 16

NC, NS = 2, 16
NT = NC * NS
CH = 128
KS = 8
SUP = CH * KS

NPAD = 10240
DUMMY = NPAD - 1
EPT = 10240
EPAD = EPT * NT
NSUP = EPT // SUP
ROWS_PER_SUB = NPAD // NS

_MESH = plsc.VectorSubcoreMesh(core_axis_name="c", subcore_axis_name="s")
_SC_PARAMS = pltpu.CompilerParams(use_tc_tiling_on_sc=False)


@functools.partial(
    pl.kernel,
    out_type=jax.ShapeDtypeStruct((NC, NPAD, DW), jnp.float32),
    mesh=_MESH,
    scratch_types=[
        pltpu.VMEM((KS, CH), jnp.int32),
        pltpu.VMEM((CH, DW), jnp.float32),
        pltpu.VMEM_SHARED((NPAD, DW), jnp.float32),
        pltpu.SemaphoreType.DMA,
    ],
    compiler_params=_SC_PARAMS,
)
def _sc_degree(col_hbm, ones_hbm, zeros_hbm, out_hbm, idxc, onesv, acc, sem):
    c = lax.axis_index("c")
    s = lax.axis_index("s")
    tid = c * NS + s
    r0 = s * ROWS_PER_SUB
    pltpu.sync_copy(zeros_hbm.at[pl.ds(r0, ROWS_PER_SUB)],
                    acc.at[pl.ds(r0, ROWS_PER_SUB)])
    pltpu.sync_copy(ones_hbm, onesv)
    plsc.subcore_barrier()

    @pl.loop(0, NSUP)
    def _(k):
        base = tid * (EPT // CH) + k * KS
        pltpu.sync_copy(col_hbm.at[pl.ds(base, KS)], idxc)
        copies = [
            pltpu.async_copy(onesv, acc.at[idxc.at[j]], sem, add=True)
            for j in range(KS)
        ]
        for cp in copies:
            cp.wait()

    plsc.subcore_barrier()
    pltpu.sync_copy(acc.at[pl.ds(r0, ROWS_PER_SUB)],
                    out_hbm.at[c].at[pl.ds(r0, ROWS_PER_SUB)])


@functools.partial(
    pl.kernel,
    out_type=jax.ShapeDtypeStruct((NC, NPAD, CP), jnp.float32),
    mesh=_MESH,
    scratch_types=[
        pltpu.VMEM((KS, CH), jnp.int32),
        pltpu.VMEM((KS, CH), jnp.int32),
        pltpu.VMEM((KS, CH, CP), jnp.float32),
        pltpu.VMEM_SHARED((NPAD, CP), jnp.float32),
        pltpu.SemaphoreType.DMA,
    ],
    compiler_params=_SC_PARAMS,
)
def _sc_hop(g_hbm, row_hbm, col_hbm, zeros_hbm, out_hbm,
            idxr, idxc, vals, acc, sem):
    c = lax.axis_index("c")
    s = lax.axis_index("s")
    tid = c * NS + s
    r0 = s * ROWS_PER_SUB
    pltpu.sync_copy(zeros_hbm.at[pl.ds(r0, ROWS_PER_SUB)],
                    acc.at[pl.ds(r0, ROWS_PER_SUB)])
    plsc.subcore_barrier()

    @pl.loop(0, NSUP)
    def _(k):
        base = tid * (EPT // CH) + k * KS
        pltpu.sync_copy(row_hbm.at[pl.ds(base, KS)], idxr)
        pltpu.sync_copy(col_hbm.at[pl.ds(base, KS)], idxc)
        gathers = [
            pltpu.async_copy(g_hbm.at[idxr.at[j]], vals.at[j], sem)
            for j in range(KS)
        ]
        for cp in gathers:
            cp.wait()
        scatters = [
            pltpu.async_copy(vals.at[j], acc.at[idxc.at[j]], sem, add=True)
            for j in range(KS)
        ]
        for cp in scatters:
            cp.wait()

    plsc.subcore_barrier()
    pltpu.sync_copy(acc.at[pl.ds(r0, ROWS_PER_SUB)],
                    out_hbm.at[c].at[pl.ds(r0, ROWS_PER_SUB)])


def _tc_matmul(xp, wt):
    def body(x_ref, w_ref, o_ref):
        o_ref[...] = jnp.dot(x_ref[...], w_ref[...],
                             preferred_element_type=jnp.float32)
    return pl.pallas_call(
        body, out_shape=jax.ShapeDtypeStruct((NPAD, CP), jnp.float32),
    )(xp, wt)


def _deg_col(d_ref):
    return d_ref[0, :, 0:1] + d_ref[1, :, 0:1] + 1.0


def _tc_scale(degp, y):
    def body(d_ref, y_ref, o_ref):
        o_ref[...] = y_ref[...] * lax.rsqrt(_deg_col(d_ref))
    return pl.pallas_call(
        body, out_shape=jax.ShapeDtypeStruct((NPAD, CP), jnp.float32),
    )(degp, y)


def _tc_mid(p, g0, degp):
    def body(p_ref, g_ref, d_ref, o_ref):
        s = p_ref[0] + p_ref[1] + g_ref[...]
        o_ref[...] = s / _deg_col(d_ref)
    return pl.pallas_call(
        body, out_shape=jax.ShapeDtypeStruct((NPAD, CP), jnp.float32),
    )(p, g0, degp)


def _tc_final(q, g1, degp, bp):
    def body(q_ref, g_ref, d_ref, b_ref, o_ref):
        t = (q_ref[0] + q_ref[1] + g_ref[...]) * lax.rsqrt(_deg_col(d_ref))
        logits = t + b_ref[...]
        mask = lax.broadcasted_iota(jnp.int32, (1, CP), 1) < C
        masked = jnp.where(mask, logits, -jnp.inf)
        m = jnp.max(masked, axis=1, keepdims=True)
        ex = jnp.where(mask, jnp.exp(logits - m), 0.0)
        lse = jnp.log(jnp.sum(ex, axis=1, keepdims=True))
        o_ref[...] = logits - m - lse
    return pl.pallas_call(
        body, out_shape=jax.ShapeDtypeStruct((NPAD, CP), jnp.float32),
    )(q, g1, degp, bp)


def kernel(x, edge_index, W, b):
    f32 = jnp.float32
    xp = jnp.zeros((NPAD, F_IN), f32).at[:N].set(x)
    wt = jnp.zeros((F_IN, CP), f32).at[:, :C].set(W.T)
    bp = jnp.zeros((1, CP), f32).at[0, :C].set(b)

    pad_r = jnp.zeros((EPAD - E,), jnp.int32)
    pad_c = jnp.full((EPAD - E,), DUMMY, jnp.int32)
    rows = jnp.concatenate([edge_index[0].astype(jnp.int32), pad_r])
    cols = jnp.concatenate([edge_index[1].astype(jnp.int32), pad_c])
    rows2d = rows.reshape(EPAD // CH, CH)
    cols2d = cols.reshape(EPAD // CH, CH)

    zeros_dw = jnp.zeros((NPAD, DW), f32)
    zeros_cp = jnp.zeros((NPAD, CP), f32)
    ones_ch = jnp.ones((CH, DW), f32)

    degp = _sc_degree(cols2d, ones_ch, zeros_dw)
    y = _tc_matmul(xp, wt)
    g0 = _tc_scale(degp, y)
    p = _sc_hop(g0, rows2d, cols2d, zeros_cp)
    g1 = _tc_mid(p, g0, degp)
    q = _sc_hop(g1, rows2d, cols2d, zeros_cp)
    out = _tc_final(q, g1, degp, bp)
    return out[:N, :C]


# speedup vs baseline: 20.3447x; 1.0901x over previous
import functools

import jax
import jax.numpy as jnp
from jax import lax
from jax.experimental import pallas as pl
from jax.experimental.pallas import tpu as pltpu
from jax.experimental.pallas import tpu_sc as plsc

N = 10000
E = 320000
F_IN = 128
C = 40
CP = 48
DW = 16

NC, NS = 2, 16
NT = NC * NS
CH = 128
KS = 5
SUP = CH * KS

NPAD = 10240
DUMMY = NPAD - 1
EPT = 10240
EPAD = EPT * NT
NSUP = EPT // SUP
NCH = EPT // CH
ROWS_PER_SUB = NPAD // NS

_MESH = plsc.VectorSubcoreMesh(core_axis_name="c", subcore_axis_name="s")
_SC_PARAMS = pltpu.CompilerParams(use_tc_tiling_on_sc=False)


@functools.partial(
    pl.kernel,
    out_type=jax.ShapeDtypeStruct((NC, NPAD, DW), jnp.float32),
    mesh=_MESH,
    scratch_types=[
        pltpu.VMEM((NCH, CH), jnp.int32),
        pltpu.VMEM((CH, DW), jnp.float32),
        pltpu.VMEM_SHARED((NPAD, DW), jnp.float32),
        pltpu.SemaphoreType.DMA,
    ],
    compiler_params=_SC_PARAMS,
)
def _sc_degree(col_hbm, ones_hbm, zeros_hbm, out_hbm, idxc, onesv, acc, sem):
    c = lax.axis_index("c")
    s = lax.axis_index("s")
    tid = c * NS + s
    r0 = s * ROWS_PER_SUB
    pltpu.sync_copy(col_hbm.at[pl.ds(tid * NCH, NCH)], idxc)
    pltpu.sync_copy(zeros_hbm.at[pl.ds(r0, ROWS_PER_SUB)],
                    acc.at[pl.ds(r0, ROWS_PER_SUB)])
    pltpu.sync_copy(ones_hbm, onesv)
    plsc.subcore_barrier()

    @pl.loop(0, NCH, step=KS)
    def _(k):
        copies = [
            pltpu.async_copy(onesv, acc.at[idxc.at[k + j]], sem, add=True)
            for j in range(KS)
        ]
        for cp in copies:
            cp.wait()

    plsc.subcore_barrier()
    pltpu.sync_copy(acc.at[pl.ds(r0, ROWS_PER_SUB)],
                    out_hbm.at[c].at[pl.ds(r0, ROWS_PER_SUB)])


@functools.partial(
    pl.kernel,
    out_type=jax.ShapeDtypeStruct((NC, NPAD, CP), jnp.float32),
    mesh=_MESH,
    scratch_types=[
        pltpu.VMEM((NCH, CH), jnp.int32),
        pltpu.VMEM((NCH, CH), jnp.int32),
        pltpu.VMEM((2, KS, CH, CP), jnp.float32),
        pltpu.VMEM_SHARED((NPAD, CP), jnp.float32),
        pltpu.SemaphoreType.DMA,
        pltpu.SemaphoreType.DMA,
    ],
    compiler_params=_SC_PARAMS,
)
def _sc_hop(g_hbm, row_hbm, col_hbm, zeros_hbm, out_hbm,
            idxr, idxc, vals, acc, gsem, ssem):
    c = lax.axis_index("c")
    s = lax.axis_index("s")
    tid = c * NS + s
    r0 = s * ROWS_PER_SUB

    pltpu.sync_copy(row_hbm.at[pl.ds(tid * NCH, NCH)], idxr)
    pltpu.sync_copy(col_hbm.at[pl.ds(tid * NCH, NCH)], idxc)
    pltpu.sync_copy(zeros_hbm.at[pl.ds(r0, ROWS_PER_SUB)],
                    acc.at[pl.ds(r0, ROWS_PER_SUB)])
    plsc.subcore_barrier()

    def issue_gathers(k, b):
        for j in range(KS):
            pltpu.async_copy(g_hbm.at[idxr.at[k * KS + j]], vals.at[b].at[j],
                             gsem)

    def wait_gathers(b):
        for j in range(KS):
            pltpu.make_async_copy(g_hbm.at[idxr.at[j]], vals.at[b].at[j],
                                  gsem).wait()

    def issue_scatters(k, b):
        for j in range(KS):
            pltpu.async_copy(vals.at[b].at[j], acc.at[idxc.at[k * KS + j]],
                             ssem, add=True)

    def wait_scatters(b):
        for j in range(KS):
            pltpu.make_async_copy(vals.at[b].at[j], acc.at[idxc.at[j]],
                                  ssem).wait()

    issue_gathers(0, 0)

    @pl.loop(0, NSUP - 1)
    def _(k):
        p = lax.rem(k, 2)
        q = 1 - p

        @pl.when(k >= 1)
        def _():
            wait_scatters(q)
        issue_gathers(k + 1, q)
        wait_gathers(p)
        issue_scatters(k, p)

    last = NSUP - 1
    pb = (NSUP - 1) % 2
    wait_scatters(1 - pb)
    wait_gathers(pb)
    issue_scatters(last, pb)
    wait_scatters(pb)

    plsc.subcore_barrier()
    pltpu.sync_copy(acc.at[pl.ds(r0, ROWS_PER_SUB)],
                    out_hbm.at[c].at[pl.ds(r0, ROWS_PER_SUB)])


def _tc_matmul(xp, wt):
    def body(x_ref, w_ref, o_ref):
        o_ref[...] = jnp.dot(x_ref[...], w_ref[...],
                             preferred_element_type=jnp.float32)
    return pl.pallas_call(
        body, out_shape=jax.ShapeDtypeStruct((NPAD, CP), jnp.float32),
    )(xp, wt)


def _deg_col(d_ref):
    return d_ref[0, :, 0:1] + d_ref[1, :, 0:1] + 1.0


def _tc_scale(degp, y):
    def body(d_ref, y_ref, o_ref):
        o_ref[...] = y_ref[...] * lax.rsqrt(_deg_col(d_ref))
    return pl.pallas_call(
        body, out_shape=jax.ShapeDtypeStruct((NPAD, CP), jnp.float32),
    )(degp, y)


def _tc_mid(p, g0, degp):
    def body(p_ref, g_ref, d_ref, o_ref):
        s = p_ref[0] + p_ref[1] + g_ref[...]
        o_ref[...] = s / _deg_col(d_ref)
    return pl.pallas_call(
        body, out_shape=jax.ShapeDtypeStruct((NPAD, CP), jnp.float32),
    )(p, g0, degp)


def _tc_final(q, g1, degp, bp):
    def body(q_ref, g_ref, d_ref, b_ref, o_ref):
        t = (q_ref[0] + q_ref[1] + g_ref[...]) * lax.rsqrt(_deg_col(d_ref))
        logits = t + b_ref[...]
        mask = lax.broadcasted_iota(jnp.int32, (1, CP), 1) < C
        masked = jnp.where(mask, logits, -jnp.inf)
        m = jnp.max(masked, axis=1, keepdims=True)
        ex = jnp.where(mask, jnp.exp(logits - m), 0.0)
        lse = jnp.log(jnp.sum(ex, axis=1, keepdims=True))
        o_ref[...] = logits - m - lse
    return pl.pallas_call(
        body, out_shape=jax.ShapeDtypeStruct((NPAD, CP), jnp.float32),
    )(q, g1, degp, bp)


def kernel(x, edge_index, W, b):
    f32 = jnp.float32
    xp = jnp.zeros((NPAD, F_IN), f32).at[:N].set(x)
    wt = jnp.zeros((F_IN, CP), f32).at[:, :C].set(W.T)
    bp = jnp.zeros((1, CP), f32).at[0, :C].set(b)

    pad_r = jnp.zeros((EPAD - E,), jnp.int32)
    pad_c = jnp.full((EPAD - E,), DUMMY, jnp.int32)
    rows = jnp.concatenate([edge_index[0].astype(jnp.int32), pad_r])
    cols = jnp.concatenate([edge_index[1].astype(jnp.int32), pad_c])
    rows2d = rows.reshape(EPAD // CH, CH)
    cols2d = cols.reshape(EPAD // CH, CH)

    zeros_dw = jnp.zeros((NPAD, DW), f32)
    zeros_cp = jnp.zeros((NPAD, CP), f32)
    ones_ch = jnp.ones((CH, DW), f32)

    degp = _sc_degree(cols2d, ones_ch, zeros_dw)
    y = _tc_matmul(xp, wt)
    g0 = _tc_scale(degp, y)
    p = _sc_hop(g0, rows2d, cols2d, zeros_cp)
    g1 = _tc_mid(p, g0, degp)
    q = _sc_hop(g1, rows2d, cols2d, zeros_cp)
    out = _tc_final(q, g1, degp, bp)
    return out[:N, :C]


# speedup vs baseline: 44.8819x; 2.2061x over previous
import functools

import jax
import jax.numpy as jnp
from jax import lax
from jax.experimental import pallas as pl
from jax.experimental.pallas import tpu as pltpu
from jax.experimental.pallas import tpu_sc as plsc

N = 10000
E = 320000
F_IN = 128
C = 40
CP = 48
DW = 16

NC, NS = 2, 16
NT = NC * NS
CH = 128
KS = 5
SUP = CH * KS

NPAD = 10240
DUMMY = NPAD - 1
EPT = 10240
EPAD = EPT * NT
NSUP = EPT // SUP
NCH = EPT // CH
ROWS_PER_SUB = NPAD // NS

_MESH = plsc.VectorSubcoreMesh(core_axis_name="c", subcore_axis_name="s")
_SC_PARAMS = pltpu.CompilerParams(use_tc_tiling_on_sc=False)


@functools.partial(
    pl.kernel,
    out_type=jax.ShapeDtypeStruct((NC, NPAD, DW), jnp.float32),
    mesh=_MESH,
    scratch_types=[
        pltpu.VMEM((NCH, CH), jnp.int32),
        pltpu.VMEM((CH, DW), jnp.float32),
        pltpu.VMEM_SHARED((NPAD, DW), jnp.float32),
        pltpu.SemaphoreType.DMA,
    ],
    compiler_params=_SC_PARAMS,
)
def _sc_degree(col_hbm, ones_hbm, zeros_hbm, out_hbm, idxc, onesv, acc, sem):
    c = lax.axis_index("c")
    s = lax.axis_index("s")
    tid = c * NS + s
    r0 = s * ROWS_PER_SUB
    pltpu.sync_copy(col_hbm.at[pl.ds(tid * NCH, NCH)], idxc)
    pltpu.sync_copy(zeros_hbm.at[pl.ds(r0, ROWS_PER_SUB)],
                    acc.at[pl.ds(r0, ROWS_PER_SUB)])
    pltpu.sync_copy(ones_hbm, onesv)
    plsc.subcore_barrier()

    @pl.loop(0, NCH, step=KS)
    def _(k):
        copies = [
            pltpu.async_copy(onesv, acc.at[idxc.at[k + j]], sem, add=True)
            for j in range(KS)
        ]
        for cp in copies:
            cp.wait()

    plsc.subcore_barrier()
    pltpu.sync_copy(acc.at[pl.ds(r0, ROWS_PER_SUB)],
                    out_hbm.at[c].at[pl.ds(r0, ROWS_PER_SUB)])


@functools.partial(
    pl.kernel,
    out_type=jax.ShapeDtypeStruct((NC, NPAD, CP), jnp.float32),
    mesh=_MESH,
    scratch_types=[
        pltpu.VMEM((NCH, CH), jnp.int32),
        pltpu.VMEM((NCH, CH), jnp.int32),
        pltpu.VMEM((2, KS, CH, CP), jnp.float32),
        pltpu.VMEM_SHARED((NPAD, CP), jnp.float32),
        pltpu.SemaphoreType.DMA,
        pltpu.SemaphoreType.DMA,
    ],
    compiler_params=_SC_PARAMS,
)
def _sc_hop(g_hbm, row_hbm, col_hbm, zeros_hbm, out_hbm,
            idxr, idxc, vals, acc, gsem, ssem):
    c = lax.axis_index("c")
    s = lax.axis_index("s")
    tid = c * NS + s
    r0 = s * ROWS_PER_SUB

    pltpu.sync_copy(row_hbm.at[pl.ds(tid * NCH, NCH)], idxr)
    pltpu.sync_copy(col_hbm.at[pl.ds(tid * NCH, NCH)], idxc)
    pltpu.sync_copy(zeros_hbm.at[pl.ds(r0, ROWS_PER_SUB)],
                    acc.at[pl.ds(r0, ROWS_PER_SUB)])
    plsc.subcore_barrier()

    def issue_gathers(k, b):
        for j in range(KS):
            pltpu.async_copy(g_hbm.at[idxr.at[k * KS + j]], vals.at[b].at[j],
                             gsem)

    def wait_gathers(b):
        for j in range(KS):
            pltpu.make_async_copy(g_hbm.at[idxr.at[j]], vals.at[b].at[j],
                                  gsem).wait()

    def issue_scatters(k, b):
        for j in range(KS):
            pltpu.async_copy(vals.at[b].at[j], acc.at[idxc.at[k * KS + j]],
                             ssem, add=True)

    def wait_scatters(b):
        for j in range(KS):
            pltpu.make_async_copy(vals.at[b].at[j], acc.at[idxc.at[j]],
                                  ssem).wait()

    issue_gathers(0, 0)

    @pl.loop(0, NSUP - 1)
    def _(k):
        p = lax.rem(k, 2)
        q = 1 - p

        @pl.when(k >= 1)
        def _():
            wait_scatters(q)
        issue_gathers(k + 1, q)
        wait_gathers(p)
        issue_scatters(k, p)

    last = NSUP - 1
    pb = (NSUP - 1) % 2
    wait_scatters(1 - pb)
    wait_gathers(pb)
    issue_scatters(last, pb)
    wait_scatters(pb)

    plsc.subcore_barrier()
    pltpu.sync_copy(acc.at[pl.ds(r0, ROWS_PER_SUB)],
                    out_hbm.at[c].at[pl.ds(r0, ROWS_PER_SUB)])


def _tc_matmul(xp, wt):
    def body(x_ref, w_ref, o_ref):
        o_ref[...] = jnp.dot(x_ref[...], w_ref[...],
                             preferred_element_type=jnp.float32)
    return pl.pallas_call(
        body, out_shape=jax.ShapeDtypeStruct((NPAD, CP), jnp.float32),
    )(xp, wt)


def _deg_col(d_ref):
    return d_ref[0, :, 0:1] + d_ref[1, :, 0:1] + 1.0


def _tc_scale(degp, y):
    def body(d_ref, y_ref, o_ref):
        o_ref[...] = y_ref[...] * lax.rsqrt(_deg_col(d_ref))
    return pl.pallas_call(
        body, out_shape=jax.ShapeDtypeStruct((NPAD, CP), jnp.float32),
    )(degp, y)


def _tc_mid(p, g0, degp):
    def body(p_ref, g_ref, d_ref, o_ref):
        s = p_ref[0] + p_ref[1] + g_ref[...]
        o_ref[...] = s / _deg_col(d_ref)
    return pl.pallas_call(
        body, out_shape=jax.ShapeDtypeStruct((NPAD, CP), jnp.float32),
    )(p, g0, degp)


def _tc_final(q, g1, degp, bp):
    def body(q_ref, g_ref, d_ref, b_ref, o_ref):
        t = (q_ref[0] + q_ref[1] + g_ref[...]) * lax.rsqrt(_deg_col(d_ref))
        logits = t + b_ref[...]
        mask = lax.broadcasted_iota(jnp.int32, (1, CP), 1) < C
        masked = jnp.where(mask, logits, -jnp.inf)
        m = jnp.max(masked, axis=1, keepdims=True)
        ex = jnp.where(mask, jnp.exp(logits - m), 0.0)
        lse = jnp.log(jnp.sum(ex, axis=1, keepdims=True))
        o_ref[...] = logits - m - lse
    return pl.pallas_call(
        body, out_shape=jax.ShapeDtypeStruct((NPAD, CP), jnp.float32),
    )(q, g1, degp, bp)


def kernel(x, edge_index, W, b):
    f32 = jnp.float32
    xp = jnp.zeros((NPAD, F_IN), f32).at[:N].set(x)
    wt = jnp.zeros((F_IN, CP), f32).at[:, :C].set(W.T)
    bp = jnp.zeros((1, CP), f32).at[0, :C].set(b)

    junk = N + jnp.arange(EPAD - E, dtype=jnp.int32) % (NPAD - N)
    pad_r = junk
    pad_c = junk
    rows = jnp.concatenate([edge_index[0].astype(jnp.int32), pad_r])
    cols = jnp.concatenate([edge_index[1].astype(jnp.int32), pad_c])
    rows2d = rows.reshape(EPAD // CH, CH)
    cols2d = cols.reshape(EPAD // CH, CH)

    zeros_dw = jnp.zeros((NPAD, DW), f32)
    zeros_cp = jnp.zeros((NPAD, CP), f32)
    ones_ch = jnp.ones((CH, DW), f32)

    degp = _sc_degree(cols2d, ones_ch, zeros_dw)
    y = _tc_matmul(xp, wt)
    g0 = _tc_scale(degp, y)
    p = _sc_hop(g0, rows2d, cols2d, zeros_cp)
    g1 = _tc_mid(p, g0, degp)
    q = _sc_hop(g1, rows2d, cols2d, zeros_cp)
    out = _tc_final(q, g1, degp, bp)
    return out[:N, :C]


# speedup vs baseline: 45.9658x; 1.0242x over previous
import functools

import numpy as np

import jax
import jax.numpy as jnp
from jax import lax
from jax.experimental import pallas as pl
from jax.experimental.pallas import tpu as pltpu
from jax.experimental.pallas import tpu_sc as plsc

N = 10000
E = 320000
F_IN = 128
C = 40
CP = 48
DW = 16

NC, NS = 2, 16
NT = NC * NS
CH = 128
KS = 5
SUP = CH * KS

NPAD = 10240
EPT = 10240
EPAD = EPT * NT
NSUP = EPT // SUP
NCH = EPT // CH
ROWS_PER_SUB = NPAD // NS

_JUNK = (N + np.arange(EPAD - E, dtype=np.int32) % (NPAD - N))
_PAD_EDGES = np.stack([_JUNK, _JUNK])

_MESH = plsc.VectorSubcoreMesh(core_axis_name="c", subcore_axis_name="s")
_SC_PARAMS = pltpu.CompilerParams(use_tc_tiling_on_sc=False)

_F32 = jnp.float32


@functools.partial(
    pl.kernel,
    out_type=(jax.ShapeDtypeStruct((NPAD, DW), _F32),
              jax.ShapeDtypeStruct((NPAD, DW), _F32)),
    mesh=_MESH,
    scratch_types=[
        pltpu.VMEM((NCH, CH), jnp.int32),
        pltpu.VMEM((CH, DW), _F32),
        pltpu.VMEM_SHARED((NPAD, DW), _F32),
        pltpu.SemaphoreType.DMA,
    ],
)
def _sc_degree(ei_hbm, ones_hbm, zeros_hbm, out0_hbm, out1_hbm,
               idxc, onesv, acc, sem):
    c = lax.axis_index("c")
    s = lax.axis_index("s")
    tid = c * NS + s
    r0 = s * ROWS_PER_SUB
    pltpu.sync_copy(ei_hbm.at[1].at[pl.ds(tid * NCH, NCH)], idxc)
    pltpu.sync_copy(zeros_hbm.at[pl.ds(r0, ROWS_PER_SUB)],
                    acc.at[pl.ds(r0, ROWS_PER_SUB)])
    pltpu.sync_copy(ones_hbm, onesv)
    plsc.subcore_barrier()

    @pl.loop(0, NCH, step=16)
    def _(k):
        for j in range(16):
            pltpu.async_copy(onesv, acc.at[idxc.at[k + j]], sem, add=True)

    @pl.loop(0, NCH)
    def _(k):
        pltpu.make_async_copy(onesv, acc.at[idxc.at[k]], sem).wait()

    plsc.subcore_barrier()

    @pl.when(c == 0)
    def _():
        pltpu.sync_copy(acc.at[pl.ds(r0, ROWS_PER_SUB)],
                        out0_hbm.at[pl.ds(r0, ROWS_PER_SUB)])

    @pl.when(c == 1)
    def _():
        pltpu.sync_copy(acc.at[pl.ds(r0, ROWS_PER_SUB)],
                        out1_hbm.at[pl.ds(r0, ROWS_PER_SUB)])


@functools.partial(
    pl.kernel,
    out_type=(jax.ShapeDtypeStruct((NPAD, CP), _F32),
              jax.ShapeDtypeStruct((NPAD, CP), _F32)),
    mesh=_MESH,
    scratch_types=[
        pltpu.VMEM((NCH, CH), jnp.int32),
        pltpu.VMEM((NCH, CH), jnp.int32),
        pltpu.VMEM((2, KS, CH, CP), _F32),
        pltpu.VMEM_SHARED((NPAD, CP), _F32),
        pltpu.SemaphoreType.DMA,
        pltpu.SemaphoreType.DMA,
    ],
    compiler_params=_SC_PARAMS,
)
def _sc_hop(g_hbm, ei_hbm, zeros_hbm, out0_hbm, out1_hbm,
            idxr, idxc, vals, acc, gsem, ssem):
    c = lax.axis_index("c")
    s = lax.axis_index("s")
    tid = c * NS + s
    r0 = s * ROWS_PER_SUB

    pltpu.sync_copy(ei_hbm.at[0].at[pl.ds(tid * NCH, NCH)], idxr)
    pltpu.sync_copy(ei_hbm.at[1].at[pl.ds(tid * NCH, NCH)], idxc)
    pltpu.sync_copy(zeros_hbm.at[pl.ds(r0, ROWS_PER_SUB)],
                    acc.at[pl.ds(r0, ROWS_PER_SUB)])
    plsc.subcore_barrier()

    def issue_gathers(k, b):
        for j in range(KS):
            pltpu.async_copy(g_hbm.at[idxr.at[k * KS + j]], vals.at[b].at[j],
                             gsem)

    def wait_gathers(b):
        for j in range(KS):
            pltpu.make_async_copy(g_hbm.at[idxr.at[j]], vals.at[b].at[j],
                                  gsem).wait()

    def issue_scatters(k, b):
        for j in range(KS):
            pltpu.async_copy(vals.at[b].at[j], acc.at[idxc.at[k * KS + j]],
                             ssem, add=True)

    def wait_scatters(b):
        for j in range(KS):
            pltpu.make_async_copy(vals.at[b].at[j], acc.at[idxc.at[j]],
                                  ssem).wait()

    issue_gathers(0, 0)

    @pl.loop(0, NSUP - 1)
    def _(k):
        p = lax.rem(k, 2)
        q = 1 - p

        @pl.when(k >= 1)
        def _():
            wait_scatters(q)
        issue_gathers(k + 1, q)
        wait_gathers(p)
        issue_scatters(k, p)

    last = NSUP - 1
    pb = (NSUP - 1) % 2
    wait_scatters(1 - pb)
    wait_gathers(pb)
    issue_scatters(last, pb)
    wait_scatters(pb)

    plsc.subcore_barrier()

    @pl.when(c == 0)
    def _():
        pltpu.sync_copy(acc.at[pl.ds(r0, ROWS_PER_SUB)],
                        out0_hbm.at[pl.ds(r0, ROWS_PER_SUB)])

    @pl.when(c == 1)
    def _():
        pltpu.sync_copy(acc.at[pl.ds(r0, ROWS_PER_SUB)],
                        out1_hbm.at[pl.ds(r0, ROWS_PER_SUB)])


def _tc_matmul(xp, wt):
    def body(x_ref, w_ref, o_ref):
        o_ref[...] = jnp.dot(x_ref[...], w_ref[...],
                             preferred_element_type=_F32)
    return pl.pallas_call(
        body, out_shape=jax.ShapeDtypeStruct((NPAD, CP), _F32),
    )(xp, wt)


def _deg_col(d0_ref, d1_ref):
    return d0_ref[:, 0:1] + d1_ref[:, 0:1] + 1.0


def _tc_scale(d0, d1, y):
    def body(d0_ref, d1_ref, y_ref, o_ref):
        o_ref[...] = y_ref[...] * lax.rsqrt(_deg_col(d0_ref, d1_ref))
    return pl.pallas_call(
        body, out_shape=jax.ShapeDtypeStruct((NPAD, CP), _F32),
    )(d0, d1, y)


def _tc_mid(p0, p1, g0, d0, d1):
    def body(p0_ref, p1_ref, g_ref, d0_ref, d1_ref, o_ref):
        ssum = p0_ref[...] + p1_ref[...] + g_ref[...]
        o_ref[...] = ssum / _deg_col(d0_ref, d1_ref)
    return pl.pallas_call(
        body, out_shape=jax.ShapeDtypeStruct((NPAD, CP), _F32),
    )(p0, p1, g0, d0, d1)


def _tc_final(q0, q1, g1, d0, d1, bp):
    def body(q0_ref, q1_ref, g_ref, d0_ref, d1_ref, b_ref, o_ref):
        t = ((q0_ref[...] + q1_ref[...] + g_ref[...])
             * lax.rsqrt(_deg_col(d0_ref, d1_ref)))
        logits = t + b_ref[...]
        mask = lax.broadcasted_iota(jnp.int32, (1, CP), 1) < C
        masked = jnp.where(mask, logits, -jnp.inf)
        m = jnp.max(masked, axis=1, keepdims=True)
        ex = jnp.where(mask, jnp.exp(logits - m), 0.0)
        lse = jnp.log(jnp.sum(ex, axis=1, keepdims=True))
        o_ref[...] = logits - m - lse
    return pl.pallas_call(
        body, out_shape=jax.ShapeDtypeStruct((NPAD, CP), _F32),
    )(q0, q1, g1, d0, d1, bp)


def kernel(x, edge_index, W, b):
    xp = jnp.zeros((NPAD, F_IN), _F32).at[:N].set(x)
    wt = jnp.zeros((F_IN, CP), _F32).at[:, :C].set(W.T)
    bp = jnp.zeros((1, CP), _F32).at[0, :C].set(b)

    ei = jnp.concatenate([edge_index.astype(jnp.int32), _PAD_EDGES], axis=1)
    ei = ei.reshape(2, EPAD // CH, CH)

    zeros_dw = jnp.zeros((NPAD, DW), _F32)
    zeros_cp = jnp.zeros((NPAD, CP), _F32)
    ones_ch = jnp.ones((CH, DW), _F32)

    d0, d1 = _sc_degree(ei, ones_ch, zeros_dw)
    y = _tc_matmul(xp, wt)
    g0 = _tc_scale(d0, d1, y)
    p0, p1 = _sc_hop(g0, ei, zeros_cp)
    g1 = _tc_mid(p0, p1, g0, d0, d1)
    q0, q1 = _sc_hop(g1, ei, zeros_cp)
    out = _tc_final(q0, q1, g1, d0, d1, bp)
    return out[:N, :C]
